# Initial kernel scaffold; baseline (speedup 1.0000x reference)
import jax
import jax.numpy as jnp
from jax.experimental import pallas as pl


def kernel(index, value, field, table, bias):
    raise NotImplementedError("write your pallas kernel here")


# speedup vs baseline: 1.0614x; 1.0614x over previous
import functools

import jax
import jax.numpy as jnp
from jax import lax
from jax.experimental import pallas as pl
from jax.experimental.pallas import tpu as pltpu, tpu_sc as plsc

B = 16384
F = 100
NC = 2
NS = 16
NW = NC * NS
RPW = B // NW
FCH = 50
NCHUNK = F // FCH
E = FCH * RPW
GROUPS = RPW // 16
UNROLL = 5


def _wide_body(idx_hbm, val_hbm, tab_hbm, bias_hbm, out_hbm,
               idx_v, val_v, g_v, out_v, bias_v, sem):
    c = lax.axis_index("c")
    s = lax.axis_index("s")
    wid = s * NC + c
    pltpu.sync_copy(bias_hbm, bias_v)
    bias_vec = bias_v[...]
    for chunk in range(NCHUNK):
        base = wid * (F * RPW) + chunk * E
        pltpu.sync_copy(idx_hbm.at[pl.ds(base, E)], idx_v)
        gather = pltpu.async_copy(tab_hbm.at[idx_v], g_v, sem)
        pltpu.sync_copy(val_hbm.at[pl.ds(base, E)], val_v)
        gather.wait()
        for g in range(GROUPS):
            goff = g * 16

            def body(it, acc, goff=goff):
                for d in range(UNROLL):
                    off = (it * UNROLL + d) * RPW + goff
                    acc = acc + g_v[pl.ds(off, 16)] * val_v[pl.ds(off, 16)]
                return acc

            init = bias_vec if chunk == 0 else out_v[pl.ds(goff, 16)]
            acc = lax.fori_loop(0, FCH // UNROLL, body, init)
            out_v[pl.ds(goff, 16)] = acc
    pltpu.sync_copy(out_v, out_hbm.at[pl.ds(wid * RPW, RPW)])


@functools.partial(jax.jit, static_argnames=())
def _wide(idx, val, tab, bias16):
    mesh = plsc.VectorSubcoreMesh(core_axis_name="c", subcore_axis_name="s",
                                  num_cores=NC, num_subcores=NS)
    return pl.kernel(
        _wide_body,
        out_type=jax.ShapeDtypeStruct((B,), jnp.float32),
        mesh=mesh,
        scratch_types=[
            pltpu.VMEM((E,), jnp.int32),
            pltpu.VMEM((E,), jnp.float32),
            pltpu.VMEM((E,), jnp.float32),
            pltpu.VMEM((RPW,), jnp.float32),
            pltpu.VMEM((16,), jnp.float32),
            pltpu.SemaphoreType.DMA,
        ],
    )(idx, val, tab, bias16)


def kernel(index, value, field, table, bias):
    del field
    idx = (index.astype(jnp.int32)
           .reshape(NW, RPW, F).transpose(0, 2, 1).reshape(-1))
    val = (value.astype(jnp.float32)
           .reshape(NW, RPW, F).transpose(0, 2, 1).reshape(-1))
    tab = table.reshape(-1)
    bias16 = jnp.broadcast_to(bias.astype(jnp.float32), (16,))
    out = _wide(idx, val, tab, bias16)
    return out.reshape(B, 1)


# speedup vs baseline: 1.1766x; 1.1085x over previous
import functools

import jax
import jax.numpy as jnp
from jax import lax
from jax.experimental import pallas as pl
from jax.experimental.pallas import tpu as pltpu, tpu_sc as plsc

B = 16384
F = 100
NC = 2
NS = 16
NW = NC * NS
RPW = B // NW
CH = 256
NCHUNK = RPW // CH
E = CH * F
R128 = E // 128
GROUPS = CH // 16
UNROLL = 4


def _wide_body(idx_hbm, val_hbm, tab_hbm, bias_hbm, out_hbm,
               idx_v, val_v, g_v, out_v, bias_v, sem):
    c = lax.axis_index("c")
    s = lax.axis_index("s")
    wid = s * NC + c
    pltpu.sync_copy(bias_hbm, bias_v)
    riota = lax.iota(jnp.int32, 16) * F
    bias_vec = bias_v[...]
    for chunk in range(NCHUNK):
        base = wid * (RPW * F) + chunk * E
        pltpu.sync_copy(idx_hbm.at[pl.ds(base, E)], idx_v)
        gather = pltpu.async_copy(tab_hbm.at[idx_v], g_v, sem)
        pltpu.sync_copy(val_hbm.at[pl.ds(base, E)], val_v)
        gather.wait()
        for g in range(GROUPS):
            gbase = g * (16 * F)

            def body(it, acc, gbase=gbase):
                for d in range(UNROLL):
                    ii = riota + (gbase + it * UNROLL + d)
                    acc = acc + plsc.load_gather(g_v, [ii]) * \
                        plsc.load_gather(val_v, [ii])
                return acc

            acc = lax.fori_loop(0, F // UNROLL, body, bias_vec)
            out_v[pl.ds(chunk * CH + g * 16, 16)] = acc
    pltpu.sync_copy(out_v, out_hbm.at[pl.ds(wid * RPW, RPW)])


@functools.partial(jax.jit, static_argnames=())
def _wide(idx, val, tab, bias16):
    mesh = plsc.VectorSubcoreMesh(core_axis_name="c", subcore_axis_name="s",
                                  num_cores=NC, num_subcores=NS)
    return pl.kernel(
        _wide_body,
        out_type=jax.ShapeDtypeStruct((B,), jnp.float32),
        mesh=mesh,
        compiler_params=pltpu.CompilerParams(needs_layout_passes=False),
        scratch_types=[
            pltpu.VMEM((E,), jnp.int32),
            pltpu.VMEM((E,), jnp.float32),
            pltpu.VMEM((E,), jnp.float32),
            pltpu.VMEM((RPW,), jnp.float32),
            pltpu.VMEM((16,), jnp.float32),
            pltpu.SemaphoreType.DMA,
        ],
    )(idx, val, tab, bias16)


def kernel(index, value, field, table, bias):
    del field
    idx = index.reshape(-1).astype(jnp.int32)
    val = value.reshape(-1).astype(jnp.float32)
    tab = table.reshape(-1)
    bias16 = jnp.broadcast_to(bias.astype(jnp.float32), (16,))
    out = _wide(idx, val, tab, bias16)
    return out.reshape(B, 1)


# speedup vs baseline: 1.1788x; 1.0019x over previous
import functools

import jax
import jax.numpy as jnp
from jax import lax
from jax.experimental import pallas as pl
from jax.experimental.pallas import tpu as pltpu, tpu_sc as plsc

B = 16384
F = 100
NC = 2
NS = 16
NW = NC * NS
RPW = B // NW
CH = 256
NCHUNK = RPW // CH
E = CH * F
R128 = E // 128
GROUPS = CH // 16
UNROLL = 4


def _wide_body(idx_hbm, val_hbm, tab_hbm, bias_hbm, out_hbm,
               idx_v, val_v, g_v, out_v, bias_v, sem):
    c = lax.axis_index("c")
    s = lax.axis_index("s")
    wid = s * NC + c
    pltpu.sync_copy(bias_hbm, bias_v)
    riota = lax.iota(jnp.int32, 16) * F
    bias_vec = bias_v[...]
    for chunk in range(NCHUNK):
        base = wid * (RPW * F) + chunk * E
        pltpu.sync_copy(idx_hbm.at[pl.ds(base, E)], idx_v)
        gather = pltpu.async_copy(tab_hbm.at[idx_v], g_v, sem)
        pltpu.sync_copy(val_hbm.at[pl.ds(base, E)], val_v)
        gather.wait()
        for g in range(GROUPS):
            gbase = g * (16 * F)

            def body(it, acc, gbase=gbase):
                for d in range(UNROLL):
                    ii = riota + (gbase + it * UNROLL + d)
                    acc = acc + plsc.load_gather(g_v, [ii]) * \
                        plsc.load_gather(val_v, [ii])
                return acc

            acc = lax.fori_loop(0, F // UNROLL, body, bias_vec)
            out_v[pl.ds(chunk * CH + g * 16, 16)] = acc
    pltpu.sync_copy(out_v, out_hbm.at[pl.ds(wid * RPW, RPW)])


@functools.partial(jax.jit, static_argnames=())
def _wide(idx, val, tab, bias16):
    mesh = plsc.VectorSubcoreMesh(core_axis_name="c", subcore_axis_name="s",
                                  num_cores=NC, num_subcores=NS)
    return pl.kernel(
        _wide_body,
        out_type=jax.ShapeDtypeStruct((B,), jnp.float32),
        mesh=mesh,
        compiler_params=pltpu.CompilerParams(needs_layout_passes=False),
        scratch_types=[
            pltpu.VMEM((E,), jnp.int32),
            pltpu.VMEM((E,), jnp.float32),
            pltpu.VMEM((E,), jnp.float32),
            pltpu.VMEM((RPW,), jnp.float32),
            pltpu.VMEM((16,), jnp.float32),
            pltpu.SemaphoreType.DMA,
        ],
    )(idx, val, tab, bias16)


def kernel(index, value, field, table, bias):
    del field
    idx = index.reshape(-1).astype(jnp.int32)
    val = value.reshape(-1).astype(jnp.float32)
    tab = lax.reshape(table, (table.shape[0],), dimensions=(1, 0))
    bias16 = jnp.broadcast_to(bias.astype(jnp.float32), (16,))
    out = _wide(idx, val, tab, bias16)
    return out.reshape(B, 1)


# speedup vs baseline: 1.5379x; 1.3047x over previous
import functools

import jax
import jax.numpy as jnp
from jax import lax
from jax.experimental import pallas as pl
from jax.experimental.pallas import tpu as pltpu, tpu_sc as plsc

B = 16384
F = 100
NC = 2
NS = 16
NW = NC * NS
RPW = B // NW
CH = 128
NCHUNK = RPW // CH
E = CH * F
R128 = E // 128
GROUPS = CH // 16
UNROLL = 4


VOCAB = 1000000
TCHUNK = 25000
NTCHUNK = VOCAB // TCHUNK


def _wide_body(idx_hbm, val_hbm, tab_hbm, bias_hbm, out_hbm,
               idx_v, val_v, g_v, out_v, bias_v, tab_sh, sem):
    c = lax.axis_index("c")
    s = lax.axis_index("s")
    wid = s * NC + c

    for k in range((NTCHUNK + NS - 1) // NS):
        t = s + k * NS

        @pl.when(t < NTCHUNK)
        def _stage_one(t=t):
            off = t * TCHUNK
            pltpu.sync_copy(tab_hbm.at[pl.ds(off, TCHUNK)],
                            g_v.at[pl.ds(0, TCHUNK)])
            pltpu.sync_copy(g_v.at[pl.ds(0, TCHUNK)],
                            tab_sh.at[pl.ds(off, TCHUNK)])

    pltpu.sync_copy(bias_hbm, bias_v)
    riota = lax.iota(jnp.int32, 16) * F
    bias_vec = bias_v[...]
    plsc.subcore_barrier()
    for chunk in range(NCHUNK):
        base = wid * (RPW * F) + chunk * E
        pltpu.sync_copy(idx_hbm.at[pl.ds(base, E)], idx_v)
        gather = pltpu.async_copy(tab_sh.at[idx_v], g_v, sem)
        pltpu.sync_copy(val_hbm.at[pl.ds(base, E)], val_v)
        gather.wait()
        for g in range(GROUPS):
            gbase = g * (16 * F)

            def body(it, acc, gbase=gbase):
                for d in range(UNROLL):
                    ii = riota + (gbase + it * UNROLL + d)
                    acc = acc + plsc.load_gather(g_v, [ii]) * \
                        plsc.load_gather(val_v, [ii])
                return acc

            acc = lax.fori_loop(0, F // UNROLL, body, bias_vec)
            out_v[pl.ds(chunk * CH + g * 16, 16)] = acc
    pltpu.sync_copy(out_v, out_hbm.at[pl.ds(wid * RPW, RPW)])


@functools.partial(jax.jit, static_argnames=())
def _wide(idx, val, tab, bias16):
    mesh = plsc.VectorSubcoreMesh(core_axis_name="c", subcore_axis_name="s",
                                  num_cores=NC, num_subcores=NS)
    return pl.kernel(
        _wide_body,
        out_type=jax.ShapeDtypeStruct((B,), jnp.float32),
        mesh=mesh,
        compiler_params=pltpu.CompilerParams(needs_layout_passes=False),
        scratch_types=[
            pltpu.VMEM((E,), jnp.int32),
            pltpu.VMEM((E,), jnp.float32),
            pltpu.VMEM((E,), jnp.float32),
            pltpu.VMEM((RPW,), jnp.float32),
            pltpu.VMEM((16,), jnp.float32),
            pltpu.VMEM_SHARED((VOCAB,), jnp.float32),
            pltpu.SemaphoreType.DMA,
        ],
    )(idx, val, tab, bias16)


def kernel(index, value, field, table, bias):
    del field
    idx = index.reshape(-1).astype(jnp.int32)
    val = value.reshape(-1).astype(jnp.float32)
    tab = lax.reshape(table, (table.shape[0],), dimensions=(1, 0))
    bias16 = jnp.broadcast_to(bias.astype(jnp.float32), (16,))
    out = _wide(idx, val, tab, bias16)
    return out.reshape(B, 1)
